# baseline (speedup 1.0000x reference)
import jax
import jax.numpy as jnp
from jax import lax
from jax.experimental import pallas as pl
from jax.experimental.pallas import tpu as pltpu
from jax.experimental.pallas import tpu_sc as plsc

N_ROWS = 8192
N_CH = 2048
NUM_CORES = 2
NUM_SUBCORES = 16
NUM_WORKERS = NUM_CORES * NUM_SUBCORES
ROWS_PER_WORKER = N_ROWS // NUM_WORKERS
RB = 8
NUM_BLOCKS = ROWS_PER_WORKER // RB
LANES = 16
CHUNKS = N_CH // LANES


def _body(x_hbm, perm_hbm, out_hbm, perm_v, in0, in1, out0, out1, out2, out3,
          sin0, sin1, sout0, sout1, sout2, sout3):
    wid = lax.axis_index("s") * NUM_CORES + lax.axis_index("c")
    base = wid * ROWS_PER_WORKER

    ins = [in0, in1]
    outs = [out0, out1, out2, out3]
    sins = [sin0, sin1]
    souts = [sout0, sout1, sout2, sout3]

    pltpu.sync_copy(perm_hbm, perm_v)

    def in_start(b, k):
        pltpu.async_copy(x_hbm.at[pl.ds(base + b * RB, RB)], ins[k], sins[k])

    def in_wait(k):
        pltpu.make_async_copy(x_hbm.at[pl.ds(base, RB)], ins[k], sins[k]).wait()

    def out_start(b, k):
        pltpu.async_copy(outs[k], out_hbm.at[pl.ds(base + b * RB, RB)], souts[k])

    def out_wait(k):
        pltpu.make_async_copy(outs[k], out_hbm.at[pl.ds(base, RB)], souts[k]).wait()

    ridxs = [jnp.full((LANES,), r, jnp.int32) for r in range(RB)]

    def permute_block(in_buf, out_buf):
        @plsc.parallel_loop(0, N_CH, LANES, unroll=8)
        def _chunk(col):
            pc = perm_v[pl.ds(col, LANES)]
            for r in range(RB):
                v = plsc.load_gather(in_buf, [ridxs[r], pc])
                out_buf[r, pl.ds(col, LANES)] = v

    in_start(0, 0)

    def outer(bb, _):
        for k in range(4):
            b = bb * 4 + k
            j = k % 2

            @pl.when(b + 1 < NUM_BLOCKS)
            def _():
                in_start(b + 1, (j + 1) % 2)

            in_wait(j)

            @pl.when(b >= 4)
            def _():
                out_wait(k)

            permute_block(ins[j], outs[k])
            out_start(b, k)
        return 0

    lax.fori_loop(0, NUM_BLOCKS // 4, outer, 0)
    for k in range(4):
        out_wait(k)


@jax.jit
def kernel(x, perm):
    mesh = plsc.VectorSubcoreMesh(core_axis_name="c", subcore_axis_name="s")
    return pl.kernel(
        _body,
        out_type=jax.ShapeDtypeStruct((N_ROWS, N_CH), jnp.float32),
        mesh=mesh,
        compiler_params=pltpu.CompilerParams(needs_layout_passes=False),
        scratch_types=[
            pltpu.VMEM((N_CH,), jnp.int32),
            pltpu.VMEM((RB, N_CH), jnp.float32),
            pltpu.VMEM((RB, N_CH), jnp.float32),
            pltpu.VMEM((RB, N_CH), jnp.float32),
            pltpu.VMEM((RB, N_CH), jnp.float32),
            pltpu.VMEM((RB, N_CH), jnp.float32),
            pltpu.VMEM((RB, N_CH), jnp.float32),
            pltpu.SemaphoreType.DMA,
            pltpu.SemaphoreType.DMA,
            pltpu.SemaphoreType.DMA,
            pltpu.SemaphoreType.DMA,
            pltpu.SemaphoreType.DMA,
            pltpu.SemaphoreType.DMA,
        ],
    )(x, perm)


# speedup vs baseline: 1.0297x; 1.0297x over previous
import jax
import jax.numpy as jnp
from jax import lax
from jax.experimental import pallas as pl
from jax.experimental.pallas import tpu as pltpu
from jax.experimental.pallas import tpu_sc as plsc

N_ROWS = 8192
N_CH = 2048
NUM_CORES = 2
NUM_SUBCORES = 16
NUM_WORKERS = NUM_CORES * NUM_SUBCORES
ROWS_PER_WORKER = N_ROWS // NUM_WORKERS
RB = 8
NUM_BLOCKS = ROWS_PER_WORKER // RB
LANES = 16
CHUNKS = N_CH // LANES


IRB = 2 * RB
NUM_IN_BLOCKS = ROWS_PER_WORKER // IRB


def _body(x_hbm, perm_hbm, out_hbm, perm_v, in0, in1, out0, out1,
          sin0, sin1, sout0, sout1):
    wid = lax.axis_index("s") * NUM_CORES + lax.axis_index("c")
    base = wid * ROWS_PER_WORKER

    ins = [in0, in1]
    outs = [out0, out1]
    sins = [sin0, sin1]
    souts = [sout0, sout1]

    pltpu.sync_copy(perm_hbm, perm_v)

    def in_start(ib, j):
        pltpu.async_copy(x_hbm.at[pl.ds(base + ib * IRB, IRB)], ins[j], sins[j])

    def in_wait(j):
        pltpu.make_async_copy(x_hbm.at[pl.ds(base, IRB)], ins[j], sins[j]).wait()

    def out_start(b, k):
        pltpu.async_copy(outs[k], out_hbm.at[pl.ds(base + b * RB, RB)], souts[k])

    def out_wait(k):
        pltpu.make_async_copy(outs[k], out_hbm.at[pl.ds(base, RB)], souts[k]).wait()

    ridxs = [jnp.full((LANES,), r, jnp.int32) for r in range(IRB)]

    def permute_block(in_buf, roff, out_buf):
        @plsc.parallel_loop(0, N_CH, LANES, unroll=8)
        def _chunk(col):
            pc = perm_v[pl.ds(col, LANES)]
            for r in range(RB):
                v = plsc.load_gather(in_buf, [ridxs[roff + r], pc])
                out_buf[r, pl.ds(col, LANES)] = v

    in_start(0, 0)
    in_start(1, 1)

    def outer(bb, _):
        b = bb * 4

        @pl.when(bb > 0)
        def _():
            in_start(2 * bb + 1, 1)

        in_wait(0)
        for half, k in ((0, 0), (1, 1)):

            @pl.when(b + half >= 2)
            def _():
                out_wait(k)

            permute_block(ins[0], half * RB, outs[k])
            out_start(b + half, k)

        @pl.when(bb + 1 < NUM_IN_BLOCKS // 2 * 2 // 2)
        def _():
            pass

        @pl.when(2 * bb + 2 < NUM_IN_BLOCKS)
        def _():
            in_start(2 * bb + 2, 0)

        in_wait(1)
        for half, k in ((0, 0), (1, 1)):
            out_wait(k)
            permute_block(ins[1], half * RB, outs[k])
            out_start(b + 2 + half, k)
        return 0

    lax.fori_loop(0, NUM_BLOCKS // 4, outer, 0)
    out_wait(0)
    out_wait(1)


@jax.jit
def kernel(x, perm):
    mesh = plsc.VectorSubcoreMesh(core_axis_name="c", subcore_axis_name="s")
    return pl.kernel(
        _body,
        out_type=jax.ShapeDtypeStruct((N_ROWS, N_CH), jnp.float32),
        mesh=mesh,
        compiler_params=pltpu.CompilerParams(needs_layout_passes=False),
        scratch_types=[
            pltpu.VMEM((N_CH,), jnp.int32),
            pltpu.VMEM((IRB, N_CH), jnp.float32),
            pltpu.VMEM((IRB, N_CH), jnp.float32),
            pltpu.VMEM((RB, N_CH), jnp.float32),
            pltpu.VMEM((RB, N_CH), jnp.float32),
            pltpu.SemaphoreType.DMA,
            pltpu.SemaphoreType.DMA,
            pltpu.SemaphoreType.DMA,
            pltpu.SemaphoreType.DMA,
        ],
    )(x, perm)


# speedup vs baseline: 1.0442x; 1.0141x over previous
import jax
import jax.numpy as jnp
from jax import lax
from jax.experimental import pallas as pl
from jax.experimental.pallas import tpu as pltpu
from jax.experimental.pallas import tpu_sc as plsc

N_ROWS = 8192
N_CH = 2048
NUM_CORES = 2
NUM_SUBCORES = 16
NUM_WORKERS = NUM_CORES * NUM_SUBCORES
ROWS_PER_WORKER = N_ROWS // NUM_WORKERS
RB = 8
NUM_BLOCKS = ROWS_PER_WORKER // RB
LANES = 16
CHUNKS = N_CH // LANES


def _body(x_hbm, perm_hbm, out_hbm, perm_v, in0, in1, in2, out0, out1, out2,
          sin0, sin1, sin2, sout0, sout1, sout2):
    wid = lax.axis_index("s") * NUM_CORES + lax.axis_index("c")
    base = wid * ROWS_PER_WORKER

    ins = [in0, in1, in2]
    outs = [out0, out1, out2]
    sins = [sin0, sin1, sin2]
    souts = [sout0, sout1, sout2]

    pltpu.sync_copy(perm_hbm, perm_v)

    def in_start(b, k):
        pltpu.async_copy(x_hbm.at[pl.ds(base + b * RB, RB)], ins[k], sins[k])

    def in_wait(k):
        pltpu.make_async_copy(x_hbm.at[pl.ds(base, RB)], ins[k], sins[k]).wait()

    def out_start(b, k):
        pltpu.async_copy(outs[k], out_hbm.at[pl.ds(base + b * RB, RB)], souts[k])

    def out_wait(k):
        pltpu.make_async_copy(outs[k], out_hbm.at[pl.ds(base, RB)], souts[k]).wait()

    ridxs = [jnp.full((LANES,), r, jnp.int32) for r in range(RB)]

    def permute_block(in_buf, out_buf):
        @plsc.parallel_loop(0, N_CH, LANES, unroll=4)
        def _chunk(col):
            pc = perm_v[pl.ds(col, LANES)]
            for r in range(RB):
                v = plsc.load_gather(in_buf, [ridxs[r], pc])
                out_buf[r, pl.ds(col, LANES)] = v

    in_start(0, 0)
    in_start(1, 1)

    def outer(bb, _):
        for k in range(3):
            b = bb * 3 + k

            @pl.when(b + 2 < NUM_BLOCKS)
            def _():
                in_start(b + 2, (k + 2) % 3)

            in_wait(k)

            @pl.when(b >= 3)
            def _():
                out_wait(k)

            permute_block(ins[k], outs[k])
            out_start(b, k)
        return 0

    lax.fori_loop(0, NUM_BLOCKS // 3, outer, 0)

    for b, k in ((30, 0), (31, 1)):
        in_wait(k)
        out_wait(k)
        permute_block(ins[k], outs[k])
        out_start(b, k)
    out_wait(2)
    out_wait(0)
    out_wait(1)


@jax.jit
def kernel(x, perm):
    mesh = plsc.VectorSubcoreMesh(core_axis_name="c", subcore_axis_name="s")
    return pl.kernel(
        _body,
        out_type=jax.ShapeDtypeStruct((N_ROWS, N_CH), jnp.float32),
        mesh=mesh,
        compiler_params=pltpu.CompilerParams(needs_layout_passes=False),
        scratch_types=[
            pltpu.VMEM((N_CH,), jnp.int32),
            pltpu.VMEM((RB, N_CH), jnp.float32),
            pltpu.VMEM((RB, N_CH), jnp.float32),
            pltpu.VMEM((RB, N_CH), jnp.float32),
            pltpu.VMEM((RB, N_CH), jnp.float32),
            pltpu.VMEM((RB, N_CH), jnp.float32),
            pltpu.VMEM((RB, N_CH), jnp.float32),
            pltpu.SemaphoreType.DMA,
            pltpu.SemaphoreType.DMA,
            pltpu.SemaphoreType.DMA,
            pltpu.SemaphoreType.DMA,
            pltpu.SemaphoreType.DMA,
            pltpu.SemaphoreType.DMA,
        ],
    )(x, perm)
